# Initial kernel scaffold; baseline (speedup 1.0000x reference)
import jax
import jax.numpy as jnp
from jax.experimental import pallas as pl


def kernel(x, Wg, bg, We, be):
    raise NotImplementedError("write your pallas kernel here")


# speedup vs baseline: 1.2133x; 1.2133x over previous
import jax
import jax.numpy as jnp
from jax.experimental import pallas as pl
from jax.experimental.pallas import tpu as pltpu


def _moe_body(x_ref, Wg_ref, bg_ref, We_ref, be_ref, out_ref, gates_ref):
    e = pl.program_id(0)

    @pl.when(e == 0)
    def _():
        logits = jnp.dot(x_ref[...], Wg_ref[...],
                         preferred_element_type=jnp.float32) + bg_ref[...]
        m = jnp.max(logits, axis=-1, keepdims=True)
        ex = jnp.exp(logits - m)
        gates_ref[...] = ex / jnp.sum(ex, axis=-1, keepdims=True)

    n_e = gates_ref.shape[1]
    onehot = (jax.lax.broadcasted_iota(jnp.int32, (1, n_e), 1) == e
              ).astype(jnp.float32)
    g = jnp.sum(gates_ref[...] * onehot, axis=1, keepdims=True)
    xg = x_ref[...] * g
    h = jnp.dot(xg, We_ref[0], preferred_element_type=jnp.float32)
    acc = h + g * be_ref[0]

    @pl.when(e == 0)
    def _():
        out_ref[...] = acc

    @pl.when(e > 0)
    def _():
        out_ref[...] = out_ref[...] + acc


def kernel(x, Wg, bg, We, be):
    T, D = x.shape
    E, _, H = We.shape
    bg2 = bg.reshape(1, E)
    be3 = be.reshape(E, 1, H)
    return pl.pallas_call(
        _moe_body,
        grid=(E,),
        in_specs=[
            pl.BlockSpec((T, D), lambda e: (0, 0)),
            pl.BlockSpec((D, E), lambda e: (0, 0)),
            pl.BlockSpec((1, E), lambda e: (0, 0)),
            pl.BlockSpec((1, D, H), lambda e: (e, 0, 0)),
            pl.BlockSpec((1, 1, H), lambda e: (e, 0, 0)),
        ],
        out_specs=pl.BlockSpec((T, H), lambda e: (0, 0)),
        out_shape=jax.ShapeDtypeStruct((T, H), jnp.float32),
        scratch_shapes=[pltpu.VMEM((T, E), jnp.float32)],
    )(x, Wg, bg2, We, be3)


# speedup vs baseline: 1.4629x; 1.2057x over previous
import jax
import jax.numpy as jnp
from jax.experimental import pallas as pl
from jax.experimental.pallas import tpu as pltpu

NBUF = 4


def _moe_body(x_ref, Wg_ref, bg_ref, We_hbm, be_ref, out_ref, wbuf, sems):
    n_experts = be_ref.shape[0]

    logits = jnp.dot(x_ref[...], Wg_ref[...],
                     preferred_element_type=jnp.float32) + bg_ref[...]
    m = jnp.max(logits, axis=-1, keepdims=True)
    ex = jnp.exp(logits - m)
    gates = ex / jnp.sum(ex, axis=-1, keepdims=True)

    def wcopy(e, slot):
        return pltpu.make_async_copy(We_hbm.at[e], wbuf.at[slot],
                                     sems.at[slot])

    for i in range(NBUF):
        wcopy(i, i).start()

    eye = jax.lax.broadcasted_iota(jnp.int32, (1, n_experts), 1)

    def step(e, _):
        slot = jax.lax.rem(e, NBUF)
        wcopy(e, slot).wait()
        g = jnp.sum(gates * (eye == e).astype(jnp.float32),
                    axis=1, keepdims=True)
        out_ref[...] += jnp.dot(x_ref[...] * g, wbuf[slot],
                                preferred_element_type=jnp.float32)

        @pl.when(e + NBUF < n_experts)
        def _():
            wcopy(e + NBUF, slot).start()

        return 0

    out_ref[...] = jnp.dot(gates, be_ref[...],
                           preferred_element_type=jnp.float32)
    jax.lax.fori_loop(0, n_experts, step, 0)


def kernel(x, Wg, bg, We, be):
    T, D = x.shape
    E, _, H = We.shape
    return pl.pallas_call(
        _moe_body,
        in_specs=[
            pl.BlockSpec(memory_space=pltpu.MemorySpace.VMEM),
            pl.BlockSpec(memory_space=pltpu.MemorySpace.VMEM),
            pl.BlockSpec(memory_space=pltpu.MemorySpace.VMEM),
            pl.BlockSpec(memory_space=pltpu.MemorySpace.HBM),
            pl.BlockSpec(memory_space=pltpu.MemorySpace.VMEM),
        ],
        out_specs=pl.BlockSpec(memory_space=pltpu.MemorySpace.VMEM),
        out_shape=jax.ShapeDtypeStruct((T, H), jnp.float32),
        scratch_shapes=[
            pltpu.VMEM((NBUF, D, H), jnp.float32),
            pltpu.SemaphoreType.DMA((NBUF,)),
        ],
    )(x, Wg, bg.reshape(1, E), We, be)
